# Initial kernel scaffold; baseline (speedup 1.0000x reference)
import jax
import jax.numpy as jnp
from jax.experimental import pallas as pl


def kernel(x, edge_index, W):
    raise NotImplementedError("write your pallas kernel here")


# speedup vs baseline: 20.7340x; 20.7340x over previous
import functools

import jax
import jax.numpy as jnp
from jax import lax
from jax.experimental import pallas as pl
from jax.experimental.pallas import tpu as pltpu
from jax.experimental.pallas import tpu_sc as plsc

NC = 2
NS = 16
LANES = 16
CHUNK = 128


def _pad_nodes(n):
  return ((n + NS * CHUNK - 1) // (NS * CHUNK)) * (NS * CHUNK)


def _sc_degree(cols, n_nodes):
  e = cols.shape[0]
  n_chunks = e // CHUNK
  n_pad = _pad_nodes(n_nodes)
  rows_per_sub = n_pad // NS
  mesh = plsc.VectorSubcoreMesh(core_axis_name="c", subcore_axis_name="s", num_cores=NC, num_subcores=NS)

  @functools.partial(
      pl.kernel,
      out_type=jax.ShapeDtypeStruct((NC, n_pad, LANES), jnp.float32),
      mesh=mesh,
      scratch_types=[
          pltpu.VMEM((CHUNK,), jnp.int32),
          pltpu.VMEM((CHUNK, LANES), jnp.float32),
          pltpu.VMEM((rows_per_sub, LANES), jnp.float32),
          pltpu.VMEM_SHARED((n_pad, LANES), jnp.float32),
      ],
      compiler_params=pltpu.CompilerParams(use_tc_tiling_on_sc=False),
  )
  def deg_kernel(cols_hbm, out_hbm, idx_v, ones_v, zero_v, acc_sh):
    c = lax.axis_index("c")
    s = lax.axis_index("s")
    wid = s * NC + c

    zero16 = jnp.zeros((LANES,), jnp.float32)
    one_row = jnp.where(lax.iota(jnp.int32, LANES) == 0, 1.0, 0.0)

    @pl.loop(0, rows_per_sub)
    def _(r):
      zero_v[r, :] = zero16

    @pl.loop(0, CHUNK)
    def _(r):
      ones_v[r, :] = one_row

    pltpu.sync_copy(zero_v, acc_sh.at[pl.ds(s * rows_per_sub, rows_per_sub)])
    plsc.subcore_barrier()

    max_rounds = (n_chunks + NC * NS - 1) // (NC * NS)

    @pl.loop(0, max_rounds)
    def _(k):
      g = wid + k * (NC * NS)

      @pl.when(g < n_chunks)
      def _():
        pltpu.sync_copy(cols_hbm.at[pl.ds(g * CHUNK, CHUNK)], idx_v)
        pltpu.sync_copy(ones_v, acc_sh.at[idx_v], add=True)

    plsc.subcore_barrier()
    pltpu.sync_copy(
        acc_sh.at[pl.ds(s * rows_per_sub, rows_per_sub)],
        out_hbm.at[c, pl.ds(s * rows_per_sub, rows_per_sub)],
    )

  return deg_kernel(cols)


def _sc_spmm(h2, src, dst, n_nodes):
  e = src.shape[0]
  d = h2.shape[1]
  n_chunks = e // CHUNK
  n_pad = _pad_nodes(n_nodes)
  rows_per_sub = n_pad // NS
  zrows = CHUNK
  mesh = plsc.VectorSubcoreMesh(core_axis_name="c", subcore_axis_name="s", num_cores=NC, num_subcores=NS)

  @functools.partial(
      pl.kernel,
      out_type=jax.ShapeDtypeStruct((NC, n_pad, d), jnp.float32),
      mesh=mesh,
      scratch_types=[
          pltpu.VMEM((CHUNK,), jnp.int32),
          pltpu.VMEM((CHUNK,), jnp.int32),
          pltpu.VMEM((CHUNK, d), jnp.float32),
          pltpu.VMEM_SHARED((n_pad, d), jnp.float32),
      ],
  )
  def spmm_kernel(h2_hbm, src_hbm, dst_hbm, out_hbm, sidx, didx, gbuf, acc_sh):
    c = lax.axis_index("c")
    s = lax.axis_index("s")
    wid = s * NC + c

    zero16 = jnp.zeros((LANES,), jnp.float32)

    @pl.loop(0, zrows)
    def _(r):
      @pl.loop(0, d, step=LANES)
      def _(j):
        gbuf[r, pl.ds(j, LANES)] = zero16

    @pl.loop(0, rows_per_sub, step=zrows)
    def _(r0):
      pltpu.sync_copy(
          gbuf.at[pl.ds(0, zrows)],
          acc_sh.at[pl.ds(s * rows_per_sub + r0, zrows)],
      )

    plsc.subcore_barrier()

    max_rounds = (n_chunks + NC * NS - 1) // (NC * NS)

    @pl.loop(0, max_rounds)
    def _(k):
      g = wid + k * (NC * NS)

      @pl.when(g < n_chunks)
      def _():
        pltpu.sync_copy(src_hbm.at[pl.ds(g * CHUNK, CHUNK)], sidx)
        pltpu.sync_copy(dst_hbm.at[pl.ds(g * CHUNK, CHUNK)], didx)
        pltpu.sync_copy(h2_hbm.at[sidx], gbuf)
        pltpu.sync_copy(gbuf, acc_sh.at[didx], add=True)

    plsc.subcore_barrier()
    pltpu.sync_copy(
        acc_sh.at[pl.ds(s * rows_per_sub, rows_per_sub)],
        out_hbm.at[c, pl.ds(s * rows_per_sub, rows_per_sub)],
    )

  return spmm_kernel(h2, src, dst)


def _tc_linear(x, w):
  n, d_in = x.shape
  d_out = w.shape[0]
  bm = 1000

  def body(x_ref, w_ref, o_ref):
    o_ref[...] = lax.dot_general(
        x_ref[...], w_ref[...],
        (((1,), (1,)), ((), ())),
        precision=lax.Precision.HIGHEST,
    )

  return pl.pallas_call(
      body,
      grid=(n // bm,),
      in_specs=[
          pl.BlockSpec((bm, d_in), lambda i: (i, 0)),
          pl.BlockSpec((d_out, d_in), lambda i: (0, 0)),
      ],
      out_specs=pl.BlockSpec((bm, d_out), lambda i: (i, 0)),
      out_shape=jax.ShapeDtypeStruct((n, d_out), jnp.float32),
  )(x, w)


def _tc_scale(h, degp):
  n, d = h.shape
  bm = 1000

  def body(h_ref, dp_ref, o_ref):
    deg = 1.0 + dp_ref[0, :, 0] + dp_ref[1, :, 0]
    o_ref[...] = h_ref[...] * lax.rsqrt(deg)[:, None]

  return pl.pallas_call(
      body,
      grid=(n // bm,),
      in_specs=[
          pl.BlockSpec((bm, d), lambda i: (i, 0)),
          pl.BlockSpec((NC, bm, LANES), lambda i: (0, i, 0)),
      ],
      out_specs=pl.BlockSpec((bm, d), lambda i: (i, 0)),
      out_shape=jax.ShapeDtypeStruct((n, d), jnp.float32),
  )(h, degp)


def _tc_combine(q, degp, h2):
  n, d = h2.shape
  bm = 1000

  def body(q_ref, dp_ref, h2_ref, o_ref):
    deg = 1.0 + dp_ref[0, :, 0] + dp_ref[1, :, 0]
    agg = q_ref[0] + q_ref[1] + h2_ref[...]
    o_ref[...] = jnp.maximum(agg * lax.rsqrt(deg)[:, None], 0.0)

  return pl.pallas_call(
      body,
      grid=(n // bm,),
      in_specs=[
          pl.BlockSpec((NC, bm, d), lambda i: (0, i, 0)),
          pl.BlockSpec((NC, bm, LANES), lambda i: (0, i, 0)),
          pl.BlockSpec((bm, d), lambda i: (i, 0)),
      ],
      out_specs=pl.BlockSpec((bm, d), lambda i: (i, 0)),
      out_shape=jax.ShapeDtypeStruct((n, d), jnp.float32),
  )(q, degp, h2)


def kernel(x, edge_index, W):
  n = x.shape[0]
  ei = edge_index.astype(jnp.int32)
  dst = ei[0]
  src = ei[1]

  degp = _sc_degree(src, n)[:, :n]
  h = _tc_linear(x, W)
  h2 = _tc_scale(h, degp)
  q = _sc_spmm(h2, src, dst, n)[:, :n]
  return _tc_combine(q, degp, h2)
